# baseline (speedup 1.0000x reference)
import functools

import jax
import jax.numpy as jnp
from jax import lax
from jax.experimental import pallas as pl
from jax.experimental.pallas import tpu as pltpu
from jax.experimental.pallas import tpu_sc as plsc

_NC = 2
_NS = 16
_NW = _NC * _NS
_L = 16

_SQRT_HALF = 0.7071067811865476


def _gelu(x):
    return 0.5 * x * (1.0 + lax.erf(x * _SQRT_HALF))


def _pre_body(z_ref, w1a_ref, b1_ref, w3b_ref, w2_ref, w3t_ref, b2_ref,
              b3_ref, a1_ref, a2_ref, w23_ref):
    z = z_ref[...]
    a1_ref[...] = (jnp.dot(z, w1a_ref[...], preferred_element_type=jnp.float32)
                   + b1_ref[...])
    bias2 = (jnp.dot(b2_ref[...], w3t_ref[...],
                     preferred_element_type=jnp.float32) + b3_ref[...])
    a2_ref[...] = (jnp.dot(z, w3b_ref[...], preferred_element_type=jnp.float32)
                   + bias2)
    w23_ref[...] = jnp.dot(w2_ref[...], w3t_ref[...],
                           preferred_element_type=jnp.float32
                           ).astype(jnp.bfloat16)


def _precompute(z, w1a, b1, w3b, w2, w3t, b2, b3):
    n, d = z.shape
    blk = 2000
    full = lambda i: (i * 0, i * 0)
    return pl.pallas_call(
        _pre_body,
        grid=(n // blk,),
        in_specs=[
            pl.BlockSpec((blk, d), lambda i: (i, i * 0)),
            pl.BlockSpec((d, d), full),
            pl.BlockSpec((1, d), full),
            pl.BlockSpec((d, d), full),
            pl.BlockSpec((d, d), full),
            pl.BlockSpec((d, d), full),
            pl.BlockSpec((1, d), full),
            pl.BlockSpec((1, d), full),
        ],
        out_specs=[
            pl.BlockSpec((blk, d), lambda i: (i, i * 0)),
            pl.BlockSpec((blk, d), lambda i: (i, i * 0)),
            pl.BlockSpec((d, d), full),
        ],
        out_shape=[
            jax.ShapeDtypeStruct((n, d), jnp.float32),
            jax.ShapeDtypeStruct((n, d), jnp.float32),
            jax.ShapeDtypeStruct((d, d), jnp.bfloat16),
        ],
    )(z, w1a, b1.reshape(1, d), w3b, w2, w3t, b2.reshape(1, d),
      b3.reshape(1, d))


def _mlp_body(a_ref, c_ref, lr_ref, w1r_ref, w23_ref, w4_ref, b4_ref,
              out_ref):
    lr = jnp.transpose(lr_ref[...])
    h = _gelu(a_ref[...] + lr * w1r_ref[...])
    g = _gelu(jnp.dot(h.astype(jnp.bfloat16), w23_ref[...],
                      preferred_element_type=jnp.float32) + c_ref[...])
    eff = jnp.sum(g * w4_ref[...], axis=1, keepdims=True) + b4_ref[...]
    out_ref[...] = jnp.transpose(jax.nn.sigmoid(eff) * jnp.abs(lr))


def _edge_mlp(ga, gb, lr2d, w1r, w23, w4r, b4):
    e, d = ga.shape
    be = 3200
    full = lambda i: (i * 0, i * 0)
    return pl.pallas_call(
        _mlp_body,
        grid=(e // be,),
        in_specs=[
            pl.BlockSpec((be, d), lambda i: (i, i * 0)),
            pl.BlockSpec((be, d), lambda i: (i, i * 0)),
            pl.BlockSpec((1, be), lambda i: (i * 0, i)),
            pl.BlockSpec((1, d), full),
            pl.BlockSpec((d, d), full),
            pl.BlockSpec((1, d), full),
            pl.BlockSpec((1, 1), full),
        ],
        out_specs=pl.BlockSpec((1, be), lambda i: (i * 0, i)),
        out_shape=jax.ShapeDtypeStruct((1, e), jnp.float32),
    )(ga, gb, lr2d, w1r, w23, w4r, b4)


def _combine_body(t_ref, o_ref):
    o_ref[...] = jnp.max(t_ref[...], axis=0)


def _combine(tables):
    w, n = tables.shape
    return pl.pallas_call(
        _combine_body,
        out_shape=jax.ShapeDtypeStruct((n,), jnp.float32),
    )(tables)


def _sc_gather(a1, a2, src, dst):
    n, d = a1.shape
    e = src.shape[0]
    epw = e // _NW
    gc = 128
    nchunk = (epw + gc - 1) // gc
    mesh = plsc.VectorSubcoreMesh(core_axis_name="c", subcore_axis_name="s")

    @functools.partial(
        pl.kernel, mesh=mesh,
        compiler_params=pltpu.CompilerParams(needs_layout_passes=False),
        out_type=[jax.ShapeDtypeStruct((e, d), jnp.float32),
                  jax.ShapeDtypeStruct((e, d), jnp.float32)],
        scratch_types=(
            [pltpu.VMEM((gc,), jnp.int32)] * 4
            + [pltpu.VMEM((gc, d), jnp.float32)] * 4
            + [pltpu.SemaphoreType.DMA] * 12
        ),
    )
    def k(a1_hbm, a2_hbm, src_hbm, dst_hbm, g1_hbm, g2_hbm,
          ia0, ia1, ib0, ib1, ba0, ba1, bb0, bb1,
          sga0, sga1, sgb0, sgb1, ssa0, ssa1, ssb0, ssb1,
          sia0, sia1, sib0, sib1):
        wid = lax.axis_index("s") * jnp.int32(_NC) + lax.axis_index("c")
        base = wid * jnp.int32(epw)
        ia, ib = [ia0, ia1], [ib0, ib1]
        ba, bb = [ba0, ba1], [bb0, bb1]
        sga, sgb = [sga0, sga1], [sgb0, sgb1]
        ssa, ssb = [ssa0, ssa1], [ssb0, ssb1]
        sia, sib = [sia0, sia1], [sib0, sib1]

        offs = [base + jnp.int32(min(ci * gc, epw - gc))
                for ci in range(nchunk)]
        idx = [None, None]
        stores = [None, None]
        idx[0] = (pltpu.async_copy(src_hbm.at[pl.ds(offs[0], gc)], ia[0],
                                   sia[0]),
                  pltpu.async_copy(dst_hbm.at[pl.ds(offs[0], gc)], ib[0],
                                   sib[0]))
        for ci in range(nchunk):
            p = ci & 1
            q = 1 - p
            idx[p][0].wait()
            idx[p][1].wait()
            if stores[p] is not None:
                stores[p][0].wait()
                stores[p][1].wait()
            cpa = pltpu.async_copy(a1_hbm.at[ia[p]], ba[p], sga[p])
            cpb = pltpu.async_copy(a2_hbm.at[ib[p]], bb[p], sgb[p])
            if ci + 1 < nchunk:
                idx[q] = (pltpu.async_copy(
                              src_hbm.at[pl.ds(offs[ci + 1], gc)], ia[q],
                              sia[q]),
                          pltpu.async_copy(
                              dst_hbm.at[pl.ds(offs[ci + 1], gc)], ib[q],
                              sib[q]))
            cpa.wait()
            cpb.wait()
            sa = pltpu.async_copy(ba[p], g1_hbm.at[pl.ds(offs[ci], gc)],
                                  ssa[p])
            sb = pltpu.async_copy(bb[p], g2_hbm.at[pl.ds(offs[ci], gc)],
                                  ssb[p])
            stores[p] = (sa, sb)
        for hs in stores:
            if hs is not None:
                hs[0].wait()
                hs[1].wait()

    return k(a1, a2, src, dst)


def _sc_segmax(dst, scores, n):
    e = dst.shape[0]
    epw = e // _NW
    mesh = plsc.VectorSubcoreMesh(core_axis_name="c", subcore_axis_name="s")

    @functools.partial(
        pl.kernel, mesh=mesh,
        compiler_params=pltpu.CompilerParams(needs_layout_passes=False),
        out_type=jax.ShapeDtypeStruct((_NW, n), jnp.float32),
        scratch_types=[
            pltpu.VMEM((epw,), jnp.int32),
            pltpu.VMEM((epw,), jnp.float32),
            pltpu.VMEM((n,), jnp.float32),
            pltpu.VMEM((_L,), jnp.int32),
            pltpu.VMEM((_L,), jnp.float32),
        ],
    )
    def k(dst_hbm, sc_hbm, tab_hbm, dst_v, val_v, tbl_v, key_s, val_s):
        wid = lax.axis_index("s") * jnp.int32(_NC) + lax.axis_index("c")
        base = wid * jnp.int32(epw)
        pltpu.sync_copy(dst_hbm.at[pl.ds(base, epw)], dst_v)
        pltpu.sync_copy(sc_hbm.at[pl.ds(base, epw)], val_v)

        zero16 = jnp.zeros((_L,), jnp.float32)

        @pl.loop(jnp.int32(0), jnp.int32(n // _L))
        def zbody(i):
            tbl_v[pl.ds(i * jnp.int32(_L), _L)] = zero16

        iota = lax.iota(jnp.int32, _L)
        rot_idx = [(iota + jnp.int32(r)) & jnp.int32(_L - 1)
                   for r in range(1, _L)]

        @pl.loop(jnp.int32(0), jnp.int32(epw // _L))
        def ebody(i):
            kk = dst_v[pl.ds(i * jnp.int32(_L), _L)]
            vv = val_v[pl.ds(i * jnp.int32(_L), _L)]
            key_s[...] = kk
            val_s[...] = vv
            cur = vv
            for idx in rot_idx:
                kr = plsc.load_gather(key_s, [idx])
                vr = plsc.load_gather(val_s, [idx])
                cur = jnp.where(kr == kk, jnp.maximum(cur, vr), cur)
            told = plsc.load_gather(tbl_v, [kk])
            plsc.store_scatter(tbl_v, [kk], jnp.maximum(told, cur))
        pltpu.sync_copy(tbl_v, tab_hbm.at[wid])

    return k(dst, scores)


def _sc_normalize(dst, scores, seg_max, n):
    e = dst.shape[0]
    epw = e // _NW
    mesh = plsc.VectorSubcoreMesh(core_axis_name="c", subcore_axis_name="s")

    @functools.partial(
        pl.kernel, mesh=mesh,
        compiler_params=pltpu.CompilerParams(needs_layout_passes=False),
        out_type=jax.ShapeDtypeStruct((e,), jnp.float32),
        scratch_types=[
            pltpu.VMEM((n,), jnp.float32),
            pltpu.VMEM((epw,), jnp.int32),
            pltpu.VMEM((epw,), jnp.float32),
            pltpu.VMEM((epw,), jnp.float32),
        ],
    )
    def k(dst_hbm, sc_hbm, max_hbm, out_hbm, tbl_v, dst_v, val_v, res_v):
        wid = lax.axis_index("s") * jnp.int32(_NC) + lax.axis_index("c")
        base = wid * jnp.int32(epw)
        pltpu.sync_copy(max_hbm, tbl_v)
        pltpu.sync_copy(dst_hbm.at[pl.ds(base, epw)], dst_v)
        pltpu.sync_copy(sc_hbm.at[pl.ds(base, epw)], val_v)

        @pl.loop(jnp.int32(0), jnp.int32(epw // _L))
        def ebody(i):
            dd = dst_v[pl.ds(i * jnp.int32(_L), _L)]
            ss = val_v[pl.ds(i * jnp.int32(_L), _L)]
            mm = plsc.load_gather(tbl_v, [dd])
            pos = mm > 0.0
            safe = jnp.where(pos, mm, 1.0)
            res_v[pl.ds(i * jnp.int32(_L), _L)] = jnp.where(pos, ss / safe, ss)
        pltpu.sync_copy(res_v, out_hbm.at[pl.ds(base, epw)])

    return k(dst, scores, seg_max)


def kernel(z, edge_index, lr_scores, fate_representation,
           W1, b1, W2, b2, W3, b3, W4, b4, W_fate, b_fate):
    n, d = z.shape
    e = edge_index.shape[1]
    src = edge_index[0].astype(jnp.int32)
    dst = edge_index[1].astype(jnp.int32)
    lr = lr_scores.astype(jnp.float32)

    w1a = W1[:d]
    w1r = W1[d:d + 1]
    w3t = W3[:d]
    w3b = W3[d:]

    a1, a2, w23 = _precompute(z.astype(jnp.float32), w1a, b1, w3b, W2, w3t,
                              b2, b3)
    w4r = W4.reshape(1, d)
    b4r = b4.reshape(1, 1)

    nch = 10
    ec = e // nch
    lr1r = lr.reshape(1, e)
    causal_parts = []
    for c in range(nch):
        s0 = c * ec
        srcc = src[s0:s0 + ec]
        dstc = dst[s0:s0 + ec]
        gac, gbc = _sc_gather(a1, a2, srcc, dstc)
        cc = _edge_mlp(gac, gbc, lr1r[:, s0:s0 + ec], w1r, w23, w4r, b4r)
        causal_parts.append(cc)
    causal = jnp.concatenate(causal_parts, axis=1).reshape(e)
    tables = _sc_segmax(dst, causal, n)
    seg_max = _combine(tables)
    out = _sc_normalize(dst, causal, seg_max, n)
    return out, jnp.zeros_like(lr)


# speedup vs baseline: 1.0183x; 1.0183x over previous
import functools

import jax
import jax.numpy as jnp
from jax import lax
from jax.experimental import pallas as pl
from jax.experimental.pallas import tpu as pltpu
from jax.experimental.pallas import tpu_sc as plsc

_NC = 2
_NS = 16
_NW = _NC * _NS
_L = 16

_SQRT_HALF = 0.7071067811865476


def _gelu(x):
    return 0.5 * x * (1.0 + lax.erf(x * _SQRT_HALF))


def _pre_body(z_ref, w1a_ref, b1_ref, w3b_ref, w2_ref, w3t_ref, b2_ref,
              b3_ref, a1_ref, a2_ref, w23_ref):
    z = z_ref[...]
    a1_ref[...] = (jnp.dot(z, w1a_ref[...], preferred_element_type=jnp.float32)
                   + b1_ref[...])
    bias2 = (jnp.dot(b2_ref[...], w3t_ref[...],
                     preferred_element_type=jnp.float32) + b3_ref[...])
    a2_ref[...] = (jnp.dot(z, w3b_ref[...], preferred_element_type=jnp.float32)
                   + bias2)
    w23_ref[...] = jnp.dot(w2_ref[...], w3t_ref[...],
                           preferred_element_type=jnp.float32
                           ).astype(jnp.bfloat16)


def _precompute(z, w1a, b1, w3b, w2, w3t, b2, b3):
    n, d = z.shape
    blk = 2000
    full = lambda i: (i * 0, i * 0)
    return pl.pallas_call(
        _pre_body,
        grid=(n // blk,),
        in_specs=[
            pl.BlockSpec((blk, d), lambda i: (i, i * 0)),
            pl.BlockSpec((d, d), full),
            pl.BlockSpec((1, d), full),
            pl.BlockSpec((d, d), full),
            pl.BlockSpec((d, d), full),
            pl.BlockSpec((d, d), full),
            pl.BlockSpec((1, d), full),
            pl.BlockSpec((1, d), full),
        ],
        out_specs=[
            pl.BlockSpec((blk, d), lambda i: (i, i * 0)),
            pl.BlockSpec((blk, d), lambda i: (i, i * 0)),
            pl.BlockSpec((d, d), full),
        ],
        out_shape=[
            jax.ShapeDtypeStruct((n, d), jnp.float32),
            jax.ShapeDtypeStruct((n, d), jnp.float32),
            jax.ShapeDtypeStruct((d, d), jnp.bfloat16),
        ],
    )(z, w1a, b1.reshape(1, d), w3b, w2, w3t, b2.reshape(1, d),
      b3.reshape(1, d))


def _mlp_body(a_ref, c_ref, lr_ref, w1r_ref, w23_ref, w4_ref, b4_ref,
              out_ref):
    lr = jnp.transpose(lr_ref[...])
    h = _gelu(a_ref[...] + lr * w1r_ref[...])
    g = _gelu(jnp.dot(h.astype(jnp.bfloat16), w23_ref[...],
                      preferred_element_type=jnp.float32) + c_ref[...])
    eff = jnp.sum(g * w4_ref[...], axis=1, keepdims=True) + b4_ref[...]
    out_ref[...] = jnp.transpose(jax.nn.sigmoid(eff) * jnp.abs(lr))


def _edge_mlp(ga, gb, lr2d, w1r, w23, w4r, b4):
    e, d = ga.shape
    be = 2560
    full = lambda i: (i * 0, i * 0)
    return pl.pallas_call(
        _mlp_body,
        grid=(e // be,),
        in_specs=[
            pl.BlockSpec((be, d), lambda i: (i, i * 0)),
            pl.BlockSpec((be, d), lambda i: (i, i * 0)),
            pl.BlockSpec((1, be), lambda i: (i * 0, i)),
            pl.BlockSpec((1, d), full),
            pl.BlockSpec((d, d), full),
            pl.BlockSpec((1, d), full),
            pl.BlockSpec((1, 1), full),
        ],
        out_specs=pl.BlockSpec((1, be), lambda i: (i * 0, i)),
        out_shape=jax.ShapeDtypeStruct((1, e), jnp.float32),
    )(ga, gb, lr2d, w1r, w23, w4r, b4)


def _combine_body(t_ref, o_ref):
    o_ref[...] = jnp.max(t_ref[...], axis=0)


def _combine(tables):
    w, n = tables.shape
    return pl.pallas_call(
        _combine_body,
        out_shape=jax.ShapeDtypeStruct((n,), jnp.float32),
    )(tables)


def _sc_gather(a1, a2, src, dst):
    n, d = a1.shape
    e = src.shape[0]
    epw = e // _NW
    gc = 128
    nchunk = (epw + gc - 1) // gc
    mesh = plsc.VectorSubcoreMesh(core_axis_name="c", subcore_axis_name="s")

    @functools.partial(
        pl.kernel, mesh=mesh,
        compiler_params=pltpu.CompilerParams(needs_layout_passes=False),
        out_type=[jax.ShapeDtypeStruct((e, d), jnp.float32),
                  jax.ShapeDtypeStruct((e, d), jnp.float32)],
        scratch_types=(
            [pltpu.VMEM((gc,), jnp.int32)] * 4
            + [pltpu.VMEM((gc, d), jnp.float32)] * 4
            + [pltpu.SemaphoreType.DMA] * 12
        ),
    )
    def k(a1_hbm, a2_hbm, src_hbm, dst_hbm, g1_hbm, g2_hbm,
          ia0, ia1, ib0, ib1, ba0, ba1, bb0, bb1,
          sga0, sga1, sgb0, sgb1, ssa0, ssa1, ssb0, ssb1,
          sia0, sia1, sib0, sib1):
        wid = lax.axis_index("s") * jnp.int32(_NC) + lax.axis_index("c")
        base = wid * jnp.int32(epw)
        ia, ib = [ia0, ia1], [ib0, ib1]
        ba, bb = [ba0, ba1], [bb0, bb1]
        sga, sgb = [sga0, sga1], [sgb0, sgb1]
        ssa, ssb = [ssa0, ssa1], [ssb0, ssb1]
        sia, sib = [sia0, sia1], [sib0, sib1]

        offs = [base + jnp.int32(min(ci * gc, epw - gc))
                for ci in range(nchunk)]
        idx = [None, None]
        stores = [None, None]
        idx[0] = (pltpu.async_copy(src_hbm.at[pl.ds(offs[0], gc)], ia[0],
                                   sia[0]),
                  pltpu.async_copy(dst_hbm.at[pl.ds(offs[0], gc)], ib[0],
                                   sib[0]))
        for ci in range(nchunk):
            p = ci & 1
            q = 1 - p
            idx[p][0].wait()
            idx[p][1].wait()
            if stores[p] is not None:
                stores[p][0].wait()
                stores[p][1].wait()
            cpa = pltpu.async_copy(a1_hbm.at[ia[p]], ba[p], sga[p])
            cpb = pltpu.async_copy(a2_hbm.at[ib[p]], bb[p], sgb[p])
            if ci + 1 < nchunk:
                idx[q] = (pltpu.async_copy(
                              src_hbm.at[pl.ds(offs[ci + 1], gc)], ia[q],
                              sia[q]),
                          pltpu.async_copy(
                              dst_hbm.at[pl.ds(offs[ci + 1], gc)], ib[q],
                              sib[q]))
            cpa.wait()
            cpb.wait()
            sa = pltpu.async_copy(ba[p], g1_hbm.at[pl.ds(offs[ci], gc)],
                                  ssa[p])
            sb = pltpu.async_copy(bb[p], g2_hbm.at[pl.ds(offs[ci], gc)],
                                  ssb[p])
            stores[p] = (sa, sb)
        for hs in stores:
            if hs is not None:
                hs[0].wait()
                hs[1].wait()

    return k(a1, a2, src, dst)


def _sc_segmax(dst, scores, n):
    e = dst.shape[0]
    epw = e // _NW
    mesh = plsc.VectorSubcoreMesh(core_axis_name="c", subcore_axis_name="s")

    @functools.partial(
        pl.kernel, mesh=mesh,
        compiler_params=pltpu.CompilerParams(needs_layout_passes=False),
        out_type=jax.ShapeDtypeStruct((_NW, n), jnp.float32),
        scratch_types=[
            pltpu.VMEM((epw,), jnp.int32),
            pltpu.VMEM((epw,), jnp.float32),
            pltpu.VMEM((n,), jnp.float32),
            pltpu.VMEM((_L,), jnp.int32),
            pltpu.VMEM((_L,), jnp.float32),
        ],
    )
    def k(dst_hbm, sc_hbm, tab_hbm, dst_v, val_v, tbl_v, key_s, val_s):
        wid = lax.axis_index("s") * jnp.int32(_NC) + lax.axis_index("c")
        base = wid * jnp.int32(epw)
        pltpu.sync_copy(dst_hbm.at[pl.ds(base, epw)], dst_v)
        pltpu.sync_copy(sc_hbm.at[pl.ds(base, epw)], val_v)

        zero16 = jnp.zeros((_L,), jnp.float32)

        @pl.loop(jnp.int32(0), jnp.int32(n // _L))
        def zbody(i):
            tbl_v[pl.ds(i * jnp.int32(_L), _L)] = zero16

        iota = lax.iota(jnp.int32, _L)
        rot_idx = [(iota + jnp.int32(r)) & jnp.int32(_L - 1)
                   for r in range(1, _L)]

        @pl.loop(jnp.int32(0), jnp.int32(epw // _L))
        def ebody(i):
            kk = dst_v[pl.ds(i * jnp.int32(_L), _L)]
            vv = val_v[pl.ds(i * jnp.int32(_L), _L)]
            key_s[...] = kk
            val_s[...] = vv
            cur = vv
            for idx in rot_idx:
                kr = plsc.load_gather(key_s, [idx])
                vr = plsc.load_gather(val_s, [idx])
                cur = jnp.where(kr == kk, jnp.maximum(cur, vr), cur)
            told = plsc.load_gather(tbl_v, [kk])
            plsc.store_scatter(tbl_v, [kk], jnp.maximum(told, cur))
        pltpu.sync_copy(tbl_v, tab_hbm.at[wid])

    return k(dst, scores)


def _sc_normalize(dst, scores, seg_max, n):
    e = dst.shape[0]
    epw = e // _NW
    mesh = plsc.VectorSubcoreMesh(core_axis_name="c", subcore_axis_name="s")

    @functools.partial(
        pl.kernel, mesh=mesh,
        compiler_params=pltpu.CompilerParams(needs_layout_passes=False),
        out_type=jax.ShapeDtypeStruct((e,), jnp.float32),
        scratch_types=[
            pltpu.VMEM((n,), jnp.float32),
            pltpu.VMEM((epw,), jnp.int32),
            pltpu.VMEM((epw,), jnp.float32),
            pltpu.VMEM((epw,), jnp.float32),
        ],
    )
    def k(dst_hbm, sc_hbm, max_hbm, out_hbm, tbl_v, dst_v, val_v, res_v):
        wid = lax.axis_index("s") * jnp.int32(_NC) + lax.axis_index("c")
        base = wid * jnp.int32(epw)
        pltpu.sync_copy(max_hbm, tbl_v)
        pltpu.sync_copy(dst_hbm.at[pl.ds(base, epw)], dst_v)
        pltpu.sync_copy(sc_hbm.at[pl.ds(base, epw)], val_v)

        @pl.loop(jnp.int32(0), jnp.int32(epw // _L))
        def ebody(i):
            dd = dst_v[pl.ds(i * jnp.int32(_L), _L)]
            ss = val_v[pl.ds(i * jnp.int32(_L), _L)]
            mm = plsc.load_gather(tbl_v, [dd])
            pos = mm > 0.0
            safe = jnp.where(pos, mm, 1.0)
            res_v[pl.ds(i * jnp.int32(_L), _L)] = jnp.where(pos, ss / safe, ss)
        pltpu.sync_copy(res_v, out_hbm.at[pl.ds(base, epw)])

    return k(dst, scores, seg_max)


def kernel(z, edge_index, lr_scores, fate_representation,
           W1, b1, W2, b2, W3, b3, W4, b4, W_fate, b_fate):
    n, d = z.shape
    e = edge_index.shape[1]
    src = edge_index[0].astype(jnp.int32)
    dst = edge_index[1].astype(jnp.int32)
    lr = lr_scores.astype(jnp.float32)

    w1a = W1[:d]
    w1r = W1[d:d + 1]
    w3t = W3[:d]
    w3b = W3[d:]

    a1, a2, w23 = _precompute(z.astype(jnp.float32), w1a, b1, w3b, W2, w3t,
                              b2, b3)
    w4r = W4.reshape(1, d)
    b4r = b4.reshape(1, 1)

    nch = 5
    ec = e // nch
    lr1r = lr.reshape(1, e)
    causal_parts = []
    for c in range(nch):
        s0 = c * ec
        srcc = src[s0:s0 + ec]
        dstc = dst[s0:s0 + ec]
        gac, gbc = _sc_gather(a1, a2, srcc, dstc)
        cc = _edge_mlp(gac, gbc, lr1r[:, s0:s0 + ec], w1r, w23, w4r, b4r)
        causal_parts.append(cc)
    causal = jnp.concatenate(causal_parts, axis=1).reshape(e)
    tables = _sc_segmax(dst, causal, n)
    seg_max = _combine(tables)
    out = _sc_normalize(dst, causal, seg_max, n)
    return out, jnp.zeros_like(lr)
